# baseline (speedup 1.0000x reference)
import functools

import jax
import jax.numpy as jnp
from jax import lax
from jax.experimental import pallas as pl
from jax.experimental.pallas import tpu as pltpu
from jax.experimental.pallas import tpu_sc as plsc


_B = 320
_NW = 32
_NR = 3


def _sc_body(n, m, d, x_hbm, y_hbm, o_hbm, bufs, gsems, ssems):
    wid = lax.axis_index("s") * 2 + lax.axis_index("c")
    full = n // _B
    nk = (full + _NW - 1) // _NW
    ylast = m // _B
    ycut = m - ylast * _B

    def off(r, clamp):
        b = wid + r * _NW
        if clamp:
            b = jnp.where(b < full, b, b - _NW)
        return b * _B

    def g_start(r, sd, src, clamp=False):
        pltpu.make_async_copy(
            src.at[pl.ds(off(r, clamp), _B)], bufs[sd], gsems[sd]).start()

    def g_wait(sd):
        pltpu.make_async_copy(
            x_hbm.at[pl.ds(0, _B)], bufs[sd], gsems[sd]).wait()

    def s_start(r, sd, clamp=False):
        pltpu.make_async_copy(
            bufs[sd], o_hbm.at[pl.ds(off(r, clamp), _B)], ssems[sd]).start()

    def s_wait(sd):
        pltpu.make_async_copy(
            bufs[sd], o_hbm.at[pl.ds(0, _B)], ssems[sd]).wait()

    g_start(0, 0, y_hbm)
    ysplit = ylast - _NW

    @pl.when(wid < ysplit)
    def _():
        g_start(1, 1, y_hbm)

    @pl.when(wid >= ysplit)
    def _():
        g_start(1, 1, x_hbm)

    for r in range(3):
        if r >= 1:
            s_wait((r + 2) % _NR)
        g_start(r + 2, (r + 2) % _NR, x_hbm)
        g_wait(r % _NR)
        s_start(r, r % _NR)

    n_steps = 30

    def step(t, carry):
        base = 3 + t * 3
        for j in range(3):
            r = base + j
            s_wait((j + 2) % _NR)
            g_start(r + 2, (j + 2) % _NR, x_hbm)
            g_wait(j % _NR)
            s_start(r, j % _NR)
        return carry

    lax.fori_loop(0, n_steps, step, 0)

    for r in range(3 + n_steps * 3, nk):
        s_wait((r + 2) % _NR)
        if r + 2 < nk:
            g_start(r + 2, (r + 2) % _NR, x_hbm, clamp=(r + 2 == nk - 1))
        g_wait(r % _NR)
        s_start(r, r % _NR, clamp=(r == nk - 1))
    s_wait((nk - 1) % _NR)

    @pl.when(wid == ysplit)
    def _():
        poff = ylast * _B
        pltpu.sync_copy(y_hbm.at[pl.ds(poff, ycut)], bufs[0].at[pl.ds(0, ycut)])
        pltpu.sync_copy(bufs[0].at[pl.ds(0, ycut)], o_hbm.at[pl.ds(poff, ycut)])


def kernel(x, index, y):
    n, d = x.shape
    m = y.shape[0]

    def body(x_hbm, y_hbm, o_hbm, *scratch):
        bufs = scratch[0:_NR]
        gsems = scratch[_NR:2 * _NR]
        ssems = scratch[2 * _NR:3 * _NR]
        _sc_body(n, m, d, x_hbm, y_hbm, o_hbm, bufs, gsems, ssems)

    sc_kernel = pl.kernel(
        body,
        out_type=jax.ShapeDtypeStruct((n, d), x.dtype),
        mesh=plsc.VectorSubcoreMesh(core_axis_name="c", subcore_axis_name="s"),
        scratch_types=(
            [pltpu.VMEM((_B, d), x.dtype)] * _NR
            + [pltpu.SemaphoreType.DMA] * (2 * _NR)
        ),
    )
    return sc_kernel(x, y)
